# Initial kernel scaffold; baseline (speedup 1.0000x reference)
import jax
import jax.numpy as jnp
from jax.experimental import pallas as pl


def kernel(X, W):
    raise NotImplementedError("write your pallas kernel here")


# speedup vs baseline: 1.2949x; 1.2949x over previous
import functools

import numpy as np
import jax
import jax.numpy as jnp
from jax import lax
from jax.experimental import pallas as pl
from jax.experimental.pallas import tpu as pltpu
from jax.experimental.pallas import tpu_sc as plsc

NUM_EMB = 8192
DIM = 32
N_TOK = 8192
TOK_BLK = 256
ARG_WIN = 2048
COMMIT = 0.1
EPS = float(np.finfo(np.float32).eps)

SC_NC = 2
SC_NS = 16
SC_NW = SC_NC * SC_NS
TOK_PER_W = N_TOK // SC_NW
CHUNK = 128
CHUNKS_PER_W = TOK_PER_W // CHUNK


def _k1_body(x_ref, w_ref, ws_ref, idx_ref):
    x = x_ref[...]
    m = lax.dot_general(x.astype(jnp.bfloat16), w_ref[...].astype(jnp.bfloat16),
                        (((1,), (1,)), ((), ())),
                        preferred_element_type=jnp.float32)
    xs = jnp.sum(x * x, axis=1, keepdims=True)
    d = (xs + ws_ref[...]) - 2.0 * m
    big = jnp.int32(np.int32(2**31 - 1))
    best_v = None
    best_i = None
    for c in range(NUM_EMB // ARG_WIN):
        dc = d[:, c * ARG_WIN:(c + 1) * ARG_WIN]
        mnc = jnp.min(dc, axis=1, keepdims=True)
        col = lax.broadcasted_iota(jnp.int32, dc.shape, 1) + jnp.int32(c * ARG_WIN)
        idxc = jnp.min(jnp.where(dc == mnc, col, big), axis=1, keepdims=True)
        mnc_r = mnc.astype(jnp.bfloat16).astype(jnp.float32)
        if best_v is None:
            best_v, best_i = mnc_r, idxc
        else:
            take = mnc < best_v
            best_v = jnp.where(take, mnc_r, best_v)
            best_i = jnp.where(take, idxc, best_i)
    idx_ref[0, 0, :] = best_i[:, 0]


def _k1_call(x2, w, ws2):
    n_blk = N_TOK // TOK_BLK
    out = pl.pallas_call(
        _k1_body,
        grid=(n_blk,),
        in_specs=[
            pl.BlockSpec((TOK_BLK, DIM), lambda i: (i, 0)),
            pl.BlockSpec((NUM_EMB, DIM), lambda i: (0, 0)),
            pl.BlockSpec((1, NUM_EMB), lambda i: (0, 0)),
        ],
        out_specs=pl.BlockSpec((1, 1, TOK_BLK), lambda i: (i, 0, 0)),
        out_shape=jax.ShapeDtypeStruct((n_blk, 1, TOK_BLK), jnp.int32),
    )(x2, w, ws2)
    return out.reshape(-1)


def _k2_body(idx_hbm, w_hbm, out_hbm, cnt_hbm,
             idx_v, rows_v, ones_v, zeros_v, hist_sh, sem):
    cid = lax.axis_index("c")
    sid = lax.axis_index("s")
    wid = sid * SC_NC + cid

    for i in range(CHUNK // 16):
        ones_v[pl.ds(i * 16, 16)] = jnp.ones((16,), jnp.float32)
    zchunk = NUM_EMB // SC_NS
    for i in range(zchunk // 16):
        zeros_v[pl.ds(i * 16, 16)] = jnp.zeros((16,), jnp.float32)
    pltpu.sync_copy(zeros_v, hist_sh.at[pl.ds(sid * zchunk, zchunk)])
    plsc.subcore_barrier()

    pltpu.sync_copy(idx_hbm.at[pl.ds(CHUNKS_PER_W * wid, CHUNKS_PER_W)], idx_v)
    for j in range(CHUNKS_PER_W):
        pltpu.async_copy(w_hbm.at[idx_v.at[j]], rows_v.at[j], sem).wait()
        pltpu.sync_copy(ones_v, hist_sh.at[idx_v.at[j]], add=True)
    pltpu.sync_copy(rows_v, out_hbm.at[pl.ds(CHUNKS_PER_W * wid, CHUNKS_PER_W)])

    plsc.subcore_barrier()

    @pl.when(sid == 0)
    def _():
        pltpu.sync_copy(hist_sh, cnt_hbm.at[cid])


def _k2_call(idx2, w):
    mesh = plsc.VectorSubcoreMesh(core_axis_name="c", subcore_axis_name="s")
    fn = functools.partial(
        pl.kernel,
        mesh=mesh,
        out_type=[
            jax.ShapeDtypeStruct((N_TOK // CHUNK, CHUNK, DIM), jnp.float32),
            jax.ShapeDtypeStruct((SC_NC, NUM_EMB), jnp.float32),
        ],
        scratch_types=[
            pltpu.VMEM((CHUNKS_PER_W, CHUNK), jnp.int32),
            pltpu.VMEM((CHUNKS_PER_W, CHUNK, DIM), jnp.float32),
            pltpu.VMEM((CHUNK,), jnp.float32),
            pltpu.VMEM((NUM_EMB // SC_NS,), jnp.float32),
            pltpu.VMEM_SHARED((NUM_EMB,), jnp.float32),
            pltpu.SemaphoreType.DMA,
        ],
        compiler_params=pltpu.CompilerParams(use_tc_tiling_on_sc=False),
    )(_k2_body)
    return fn(idx2, w)


def _k3_body(x_ref, q_ref, cnt_ref, loss_ref, perp_ref):
    diff = q_ref[...] - x_ref[...]
    m = jnp.sum(diff * diff) * np.float32(1.0 / (N_TOK * DIM))
    loss_ref[...] = jnp.reshape(m + np.float32(COMMIT) * m, (1, 1))
    p = (cnt_ref[0:1, :] + cnt_ref[1:2, :]) * np.float32(1.0 / N_TOK)
    ent = jnp.sum(p * jnp.log(p + np.float32(EPS)))
    perp_ref[...] = jnp.reshape(jnp.exp(-ent), (1, 1))


def _k3_call(x2, q2, cnt):
    return pl.pallas_call(
        _k3_body,
        out_shape=[
            jax.ShapeDtypeStruct((1, 1), jnp.float32),
            jax.ShapeDtypeStruct((1, 1), jnp.float32),
        ],
    )(x2, q2, cnt)


def kernel(X, W):
    x2 = X.reshape(-1, DIM)
    ws2 = jnp.sum(W ** 2, axis=1).reshape(1, NUM_EMB)
    enc = _k1_call(x2, W, ws2)
    idx2 = enc.reshape(N_TOK // CHUNK, CHUNK)
    q_blocks, cnt = _k2_call(idx2, W)
    q2 = q_blocks.reshape(N_TOK, DIM)
    loss, perp = _k3_call(x2, q2, cnt)
    quantized_st = (x2 + (q2 - x2)).reshape(X.shape)
    enc_idx = enc.reshape(X.shape[0], -1)
    return quantized_st, enc_idx, perp.reshape(()), loss.reshape(())


# speedup vs baseline: 1.3775x; 1.0638x over previous
import functools

import numpy as np
import jax
import jax.numpy as jnp
from jax import lax
from jax.experimental import pallas as pl
from jax.experimental.pallas import tpu as pltpu
from jax.experimental.pallas import tpu_sc as plsc

NUM_EMB = 8192
DIM = 32
N_TOK = 8192
TOK_BLK = 256
ARG_WIN = 2048
STRIDE = 128
COMMIT = 0.1
EPS = float(np.finfo(np.float32).eps)

SC_NC = 2
SC_NS = 16
SC_NW = SC_NC * SC_NS
TOK_PER_W = N_TOK // SC_NW
CHUNK = 128
CHUNKS_PER_W = TOK_PER_W // CHUNK


def _k1_body(x_ref, w_ref, ws_ref, idx_ref):
    x = x_ref[...]
    m = lax.dot_general(x.astype(jnp.bfloat16), w_ref[...].astype(jnp.bfloat16),
                        (((1,), (1,)), ((), ())),
                        preferred_element_type=jnp.float32)
    xs = jnp.sum(x * x, axis=1, keepdims=True)
    ws = ws_ref[...]
    big = jnp.int32(np.int32(2**31 - 1))
    lane_col = lax.broadcasted_iota(jnp.int32, (TOK_BLK, STRIDE), 1)
    best_v = None
    best_i = None
    for c in range(NUM_EMB // ARG_WIN):
        run_v = None
        run_i = None
        for s in range(ARG_WIN // STRIDE):
            j0 = c * ARG_WIN + s * STRIDE
            ds = (xs + ws[:, j0:j0 + STRIDE]) - 2.0 * m[:, j0:j0 + STRIDE]
            if run_v is None:
                run_v = ds
                run_i = lane_col + jnp.int32(j0)
            else:
                take = ds < run_v
                run_v = jnp.where(take, ds, run_v)
                run_i = jnp.where(take, lane_col + jnp.int32(j0), run_i)
        mnc = jnp.min(run_v, axis=1, keepdims=True)
        idxc = jnp.min(jnp.where(run_v == mnc, run_i, big), axis=1, keepdims=True)
        mnc_r = mnc.astype(jnp.bfloat16).astype(jnp.float32)
        if best_v is None:
            best_v, best_i = mnc_r, idxc
        else:
            take = mnc < best_v
            best_v = jnp.where(take, mnc_r, best_v)
            best_i = jnp.where(take, idxc, best_i)
    idx_ref[0, 0, :] = best_i[:, 0]


def _k1_call(x2, w, ws2):
    n_blk = N_TOK // TOK_BLK
    out = pl.pallas_call(
        _k1_body,
        grid=(n_blk,),
        in_specs=[
            pl.BlockSpec((TOK_BLK, DIM), lambda i: (i, 0)),
            pl.BlockSpec((NUM_EMB, DIM), lambda i: (0, 0)),
            pl.BlockSpec((1, NUM_EMB), lambda i: (0, 0)),
        ],
        out_specs=pl.BlockSpec((1, 1, TOK_BLK), lambda i: (i, 0, 0)),
        out_shape=jax.ShapeDtypeStruct((n_blk, 1, TOK_BLK), jnp.int32),
    )(x2, w, ws2)
    return out.reshape(-1)


def _k2_body(idx_hbm, w_hbm, out_hbm, cnt_hbm,
             idx_v, rows_v, ones_v, zeros_v, hist_sh, sem):
    cid = lax.axis_index("c")
    sid = lax.axis_index("s")
    wid = sid * SC_NC + cid

    for i in range(CHUNK // 16):
        ones_v[pl.ds(i * 16, 16)] = jnp.ones((16,), jnp.float32)
    zchunk = NUM_EMB // SC_NS
    for i in range(zchunk // 16):
        zeros_v[pl.ds(i * 16, 16)] = jnp.zeros((16,), jnp.float32)
    pltpu.sync_copy(zeros_v, hist_sh.at[pl.ds(sid * zchunk, zchunk)])
    plsc.subcore_barrier()

    pltpu.sync_copy(idx_hbm.at[pl.ds(CHUNKS_PER_W * wid, CHUNKS_PER_W)], idx_v)
    for j in range(CHUNKS_PER_W):
        pltpu.async_copy(w_hbm.at[idx_v.at[j]], rows_v.at[j], sem).wait()
        pltpu.sync_copy(ones_v, hist_sh.at[idx_v.at[j]], add=True)
    pltpu.sync_copy(rows_v, out_hbm.at[pl.ds(CHUNKS_PER_W * wid, CHUNKS_PER_W)])

    plsc.subcore_barrier()

    @pl.when(sid == 0)
    def _():
        pltpu.sync_copy(hist_sh, cnt_hbm.at[cid])


def _k2_call(idx2, w):
    mesh = plsc.VectorSubcoreMesh(core_axis_name="c", subcore_axis_name="s")
    fn = functools.partial(
        pl.kernel,
        mesh=mesh,
        out_type=[
            jax.ShapeDtypeStruct((N_TOK // CHUNK, CHUNK, DIM), jnp.float32),
            jax.ShapeDtypeStruct((SC_NC, NUM_EMB), jnp.float32),
        ],
        scratch_types=[
            pltpu.VMEM((CHUNKS_PER_W, CHUNK), jnp.int32),
            pltpu.VMEM((CHUNKS_PER_W, CHUNK, DIM), jnp.float32),
            pltpu.VMEM((CHUNK,), jnp.float32),
            pltpu.VMEM((NUM_EMB // SC_NS,), jnp.float32),
            pltpu.VMEM_SHARED((NUM_EMB,), jnp.float32),
            pltpu.SemaphoreType.DMA,
        ],
        compiler_params=pltpu.CompilerParams(use_tc_tiling_on_sc=False),
    )(_k2_body)
    return fn(idx2, w)


def _k3_body(x_ref, q_ref, cnt_ref, loss_ref, perp_ref):
    diff = q_ref[...] - x_ref[...]
    m = jnp.sum(diff * diff) * np.float32(1.0 / (N_TOK * DIM))
    loss_ref[...] = jnp.reshape(m + np.float32(COMMIT) * m, (1, 1))
    p = (cnt_ref[0:1, :] + cnt_ref[1:2, :]) * np.float32(1.0 / N_TOK)
    ent = jnp.sum(p * jnp.log(p + np.float32(EPS)))
    perp_ref[...] = jnp.reshape(jnp.exp(-ent), (1, 1))


def _k3_call(x2, q2, cnt):
    return pl.pallas_call(
        _k3_body,
        out_shape=[
            jax.ShapeDtypeStruct((1, 1), jnp.float32),
            jax.ShapeDtypeStruct((1, 1), jnp.float32),
        ],
    )(x2, q2, cnt)


def kernel(X, W):
    x2 = X.reshape(-1, DIM)
    ws2 = jnp.sum(W ** 2, axis=1).reshape(1, NUM_EMB)
    enc = _k1_call(x2, W, ws2)
    idx2 = enc.reshape(N_TOK // CHUNK, CHUNK)
    q_blocks, cnt = _k2_call(idx2, W)
    q2 = q_blocks.reshape(N_TOK, DIM)
    loss, perp = _k3_call(x2, q2, cnt)
    quantized_st = (x2 + (q2 - x2)).reshape(X.shape)
    enc_idx = enc.reshape(X.shape[0], -1)
    return quantized_st, enc_idx, perp.reshape(()), loss.reshape(())


# speedup vs baseline: 1.5079x; 1.0946x over previous
import functools

import numpy as np
import jax
import jax.numpy as jnp
from jax import lax
from jax.experimental import pallas as pl
from jax.experimental.pallas import tpu as pltpu
from jax.experimental.pallas import tpu_sc as plsc

NUM_EMB = 8192
DIM = 32
N_TOK = 8192
TOK_BLK = 512
ARG_WIN = 2048
STRIDE = 128
COMMIT = 0.1
EPS = float(np.finfo(np.float32).eps)

SC_NC = 2
SC_NS = 16
SC_NW = SC_NC * SC_NS
TOK_PER_W = N_TOK // SC_NW
CHUNK = 128
CHUNKS_PER_W = TOK_PER_W // CHUNK


def _k1_body(x_ref, w_ref, ws_ref, idx_ref):
    x = x_ref[...]
    m = lax.dot_general(x.astype(jnp.bfloat16), w_ref[...].astype(jnp.bfloat16),
                        (((1,), (1,)), ((), ())),
                        preferred_element_type=jnp.float32)
    xs = jnp.sum(x * x, axis=1, keepdims=True)
    ws = ws_ref[...]
    big = jnp.int32(np.int32(2**31 - 1))
    lane_col = lax.broadcasted_iota(jnp.int32, (TOK_BLK, STRIDE), 1)
    best_v = None
    best_i = None
    for c in range(NUM_EMB // ARG_WIN):
        run_v = None
        run_i = None
        for s in range(ARG_WIN // STRIDE):
            j0 = c * ARG_WIN + s * STRIDE
            ds = (xs + ws[:, j0:j0 + STRIDE]) - 2.0 * m[:, j0:j0 + STRIDE]
            if run_v is None:
                run_v = ds
                run_i = lane_col + jnp.int32(j0)
            else:
                take = ds < run_v
                run_v = jnp.where(take, ds, run_v)
                run_i = jnp.where(take, lane_col + jnp.int32(j0), run_i)
        mnc = jnp.min(run_v, axis=1, keepdims=True)
        idxc = jnp.min(jnp.where(run_v == mnc, run_i, big), axis=1, keepdims=True)
        mnc_r = mnc.astype(jnp.bfloat16).astype(jnp.float32)
        if best_v is None:
            best_v, best_i = mnc_r, idxc
        else:
            take = mnc < best_v
            best_v = jnp.where(take, mnc_r, best_v)
            best_i = jnp.where(take, idxc, best_i)
    idx_ref[0, 0, :] = best_i[:, 0]


def _k1_call(x2, w, ws2):
    n_blk = N_TOK // TOK_BLK
    out = pl.pallas_call(
        _k1_body,
        grid=(n_blk,),
        in_specs=[
            pl.BlockSpec((TOK_BLK, DIM), lambda i: (i, 0)),
            pl.BlockSpec((NUM_EMB, DIM), lambda i: (0, 0)),
            pl.BlockSpec((1, NUM_EMB), lambda i: (0, 0)),
        ],
        out_specs=pl.BlockSpec((1, 1, TOK_BLK), lambda i: (i, 0, 0)),
        out_shape=jax.ShapeDtypeStruct((n_blk, 1, TOK_BLK), jnp.int32),
    )(x2, w, ws2)
    return out.reshape(-1)


def _k2_body(idx_hbm, w_hbm, out_hbm, cnt_hbm,
             idx_v, rows_v, ones_v, zeros_v, hist_sh, sem):
    cid = lax.axis_index("c")
    sid = lax.axis_index("s")
    wid = sid * SC_NC + cid

    for i in range(CHUNK // 16):
        ones_v[pl.ds(i * 16, 16)] = jnp.ones((16,), jnp.float32)
    zchunk = NUM_EMB // SC_NS
    for i in range(zchunk // 16):
        zeros_v[pl.ds(i * 16, 16)] = jnp.zeros((16,), jnp.float32)
    pltpu.sync_copy(zeros_v, hist_sh.at[pl.ds(sid * zchunk, zchunk)])
    plsc.subcore_barrier()

    pltpu.sync_copy(idx_hbm.at[pl.ds(CHUNKS_PER_W * wid, CHUNKS_PER_W)], idx_v)
    for j in range(CHUNKS_PER_W):
        pltpu.async_copy(w_hbm.at[idx_v.at[j]], rows_v.at[j], sem).wait()
        pltpu.sync_copy(ones_v, hist_sh.at[idx_v.at[j]], add=True)
    pltpu.sync_copy(rows_v, out_hbm.at[pl.ds(CHUNKS_PER_W * wid, CHUNKS_PER_W)])

    plsc.subcore_barrier()

    @pl.when(sid == 0)
    def _():
        pltpu.sync_copy(hist_sh, cnt_hbm.at[cid])


def _k2_call(idx2, w):
    mesh = plsc.VectorSubcoreMesh(core_axis_name="c", subcore_axis_name="s")
    fn = functools.partial(
        pl.kernel,
        mesh=mesh,
        out_type=[
            jax.ShapeDtypeStruct((N_TOK // CHUNK, CHUNK, DIM), jnp.float32),
            jax.ShapeDtypeStruct((SC_NC, NUM_EMB), jnp.float32),
        ],
        scratch_types=[
            pltpu.VMEM((CHUNKS_PER_W, CHUNK), jnp.int32),
            pltpu.VMEM((CHUNKS_PER_W, CHUNK, DIM), jnp.float32),
            pltpu.VMEM((CHUNK,), jnp.float32),
            pltpu.VMEM((NUM_EMB // SC_NS,), jnp.float32),
            pltpu.VMEM_SHARED((NUM_EMB,), jnp.float32),
            pltpu.SemaphoreType.DMA,
        ],
        compiler_params=pltpu.CompilerParams(use_tc_tiling_on_sc=False),
    )(_k2_body)
    return fn(idx2, w)


def _k3_body(x_ref, q_ref, cnt_ref, qst_ref, loss_ref, perp_ref):
    x = x_ref[...]
    q = q_ref[...]
    diff = q - x
    qst_ref[...] = x + diff
    m = jnp.sum(diff * diff) * np.float32(1.0 / (N_TOK * DIM))
    loss_ref[...] = jnp.reshape(m + np.float32(COMMIT) * m, (1, 1))
    p = (cnt_ref[0:1, :] + cnt_ref[1:2, :]) * np.float32(1.0 / N_TOK)
    ent = jnp.sum(p * jnp.log(p + np.float32(EPS)))
    perp_ref[...] = jnp.reshape(jnp.exp(-ent), (1, 1))


def _k3_call(x2, q2, cnt):
    return pl.pallas_call(
        _k3_body,
        out_shape=[
            jax.ShapeDtypeStruct((N_TOK, DIM), jnp.float32),
            jax.ShapeDtypeStruct((1, 1), jnp.float32),
            jax.ShapeDtypeStruct((1, 1), jnp.float32),
        ],
    )(x2, q2, cnt)


def kernel(X, W):
    x2 = X.reshape(-1, DIM)
    ws2 = jnp.sum(W ** 2, axis=1).reshape(1, NUM_EMB)
    enc = _k1_call(x2, W, ws2)
    idx2 = enc.reshape(N_TOK // CHUNK, CHUNK)
    q_blocks, cnt = _k2_call(idx2, W)
    q2 = q_blocks.reshape(N_TOK, DIM)
    qst, loss, perp = _k3_call(x2, q2, cnt)
    quantized_st = qst.reshape(X.shape)
    enc_idx = enc.reshape(X.shape[0], -1)
    return quantized_st, enc_idx, perp.reshape(()), loss.reshape(())


# speedup vs baseline: 1.5209x; 1.0086x over previous
import functools

import numpy as np
import jax
import jax.numpy as jnp
from jax import lax
from jax.experimental import pallas as pl
from jax.experimental.pallas import tpu as pltpu
from jax.experimental.pallas import tpu_sc as plsc

NUM_EMB = 8192
DIM = 32
N_TOK = 8192
TOK_BLK = 1024
ARG_WIN = 2048
STRIDE = 128
COMMIT = 0.1
EPS = float(np.finfo(np.float32).eps)

SC_NC = 2
SC_NS = 16
SC_NW = SC_NC * SC_NS
TOK_PER_W = N_TOK // SC_NW
CHUNK = 128
CHUNKS_PER_W = TOK_PER_W // CHUNK


def _k1_body(x_ref, w_ref, ws_ref, idx_ref):
    x = x_ref[...]
    m = lax.dot_general(x.astype(jnp.bfloat16), w_ref[...].astype(jnp.bfloat16),
                        (((1,), (1,)), ((), ())),
                        preferred_element_type=jnp.float32)
    xs = jnp.sum(x * x, axis=1, keepdims=True)
    ws = ws_ref[...]
    big = jnp.int32(np.int32(2**31 - 1))
    lane_col = lax.broadcasted_iota(jnp.int32, (TOK_BLK, STRIDE), 1)
    best_v = None
    best_i = None
    for c in range(NUM_EMB // ARG_WIN):
        run_v = None
        run_i = None
        for s in range(ARG_WIN // STRIDE):
            j0 = c * ARG_WIN + s * STRIDE
            ds = (xs + ws[:, j0:j0 + STRIDE]) - 2.0 * m[:, j0:j0 + STRIDE]
            if run_v is None:
                run_v = ds
                run_i = lane_col + jnp.int32(j0)
            else:
                take = ds < run_v
                run_v = jnp.where(take, ds, run_v)
                run_i = jnp.where(take, lane_col + jnp.int32(j0), run_i)
        mnc = jnp.min(run_v, axis=1, keepdims=True)
        idxc = jnp.min(jnp.where(run_v == mnc, run_i, big), axis=1, keepdims=True)
        mnc_r = mnc.astype(jnp.bfloat16).astype(jnp.float32)
        if best_v is None:
            best_v, best_i = mnc_r, idxc
        else:
            take = mnc < best_v
            best_v = jnp.where(take, mnc_r, best_v)
            best_i = jnp.where(take, idxc, best_i)
    idx_ref[0, 0, :] = best_i[:, 0]


def _k1_call(x2, w, ws2):
    n_blk = N_TOK // TOK_BLK
    out = pl.pallas_call(
        _k1_body,
        grid=(n_blk,),
        in_specs=[
            pl.BlockSpec((TOK_BLK, DIM), lambda i: (i, 0)),
            pl.BlockSpec((NUM_EMB, DIM), lambda i: (0, 0)),
            pl.BlockSpec((1, NUM_EMB), lambda i: (0, 0)),
        ],
        out_specs=pl.BlockSpec((1, 1, TOK_BLK), lambda i: (i, 0, 0)),
        out_shape=jax.ShapeDtypeStruct((n_blk, 1, TOK_BLK), jnp.int32),
    )(x2, w, ws2)
    return out.reshape(-1)


def _k2_body(idx_hbm, w_hbm, out_hbm, cnt_hbm,
             idx_v, rows_v, ones_v, zeros_v, hist_sh, sem):
    cid = lax.axis_index("c")
    sid = lax.axis_index("s")
    wid = sid * SC_NC + cid

    for i in range(CHUNK // 16):
        ones_v[pl.ds(i * 16, 16)] = jnp.ones((16,), jnp.float32)
    zchunk = NUM_EMB // SC_NS
    for i in range(zchunk // 16):
        zeros_v[pl.ds(i * 16, 16)] = jnp.zeros((16,), jnp.float32)
    pltpu.sync_copy(zeros_v, hist_sh.at[pl.ds(sid * zchunk, zchunk)])
    plsc.subcore_barrier()

    pltpu.sync_copy(idx_hbm.at[pl.ds(CHUNKS_PER_W * wid, CHUNKS_PER_W)], idx_v)
    for j in range(CHUNKS_PER_W):
        pltpu.async_copy(w_hbm.at[idx_v.at[j]], rows_v.at[j], sem).wait()
        pltpu.sync_copy(ones_v, hist_sh.at[idx_v.at[j]], add=True)
    pltpu.sync_copy(rows_v, out_hbm.at[pl.ds(CHUNKS_PER_W * wid, CHUNKS_PER_W)])

    plsc.subcore_barrier()

    @pl.when(sid == 0)
    def _():
        pltpu.sync_copy(hist_sh, cnt_hbm.at[cid])


def _k2_call(idx2, w):
    mesh = plsc.VectorSubcoreMesh(core_axis_name="c", subcore_axis_name="s")
    fn = functools.partial(
        pl.kernel,
        mesh=mesh,
        out_type=[
            jax.ShapeDtypeStruct((N_TOK // CHUNK, CHUNK, DIM), jnp.float32),
            jax.ShapeDtypeStruct((SC_NC, NUM_EMB), jnp.float32),
        ],
        scratch_types=[
            pltpu.VMEM((CHUNKS_PER_W, CHUNK), jnp.int32),
            pltpu.VMEM((CHUNKS_PER_W, CHUNK, DIM), jnp.float32),
            pltpu.VMEM((CHUNK,), jnp.float32),
            pltpu.VMEM((NUM_EMB // SC_NS,), jnp.float32),
            pltpu.VMEM_SHARED((NUM_EMB,), jnp.float32),
            pltpu.SemaphoreType.DMA,
        ],
        compiler_params=pltpu.CompilerParams(use_tc_tiling_on_sc=False),
    )(_k2_body)
    return fn(idx2, w)


def _k3_body(x_ref, q_ref, cnt_ref, qst_ref, loss_ref, perp_ref):
    x = x_ref[...]
    q = q_ref[...]
    diff = q - x
    qst_ref[...] = x + diff
    m = jnp.sum(diff * diff) * np.float32(1.0 / (N_TOK * DIM))
    loss_ref[...] = jnp.reshape(m + np.float32(COMMIT) * m, (1, 1))
    p = (cnt_ref[0:1, :] + cnt_ref[1:2, :]) * np.float32(1.0 / N_TOK)
    ent = jnp.sum(p * jnp.log(p + np.float32(EPS)))
    perp_ref[...] = jnp.reshape(jnp.exp(-ent), (1, 1))


def _k3_call(x2, q2, cnt):
    return pl.pallas_call(
        _k3_body,
        out_shape=[
            jax.ShapeDtypeStruct((N_TOK, DIM), jnp.float32),
            jax.ShapeDtypeStruct((1, 1), jnp.float32),
            jax.ShapeDtypeStruct((1, 1), jnp.float32),
        ],
    )(x2, q2, cnt)


def kernel(X, W):
    x2 = X.reshape(-1, DIM)
    ws2 = jnp.sum(W ** 2, axis=1).reshape(1, NUM_EMB)
    enc = _k1_call(x2, W, ws2)
    idx2 = enc.reshape(N_TOK // CHUNK, CHUNK)
    q_blocks, cnt = _k2_call(idx2, W)
    q2 = q_blocks.reshape(N_TOK, DIM)
    qst, loss, perp = _k3_call(x2, q2, cnt)
    quantized_st = qst.reshape(X.shape)
    enc_idx = enc.reshape(X.shape[0], -1)
    return quantized_st, enc_idx, perp.reshape(()), loss.reshape(())


# speedup vs baseline: 1.6068x; 1.0565x over previous
import functools

import numpy as np
import jax
import jax.numpy as jnp
from jax import lax
from jax.experimental import pallas as pl
from jax.experimental.pallas import tpu as pltpu
from jax.experimental.pallas import tpu_sc as plsc

NUM_EMB = 8192
DIM = 32
N_TOK = 8192
TOK_BLK = 1024
ARG_WIN = 2048
STRIDE = 128
COMMIT = 0.1
EPS = float(np.finfo(np.float32).eps)

SC_NC = 2
SC_NS = 16
SC_NW = SC_NC * SC_NS
TOK_PER_W = N_TOK // SC_NW
CHUNK = 128
CHUNKS_PER_W = TOK_PER_W // CHUNK


def _k1_body(x_ref, w_ref, ws_ref, idx_ref):
    x = x_ref[...]
    m2 = lax.dot_general((x * np.float32(-2.0)).astype(jnp.bfloat16),
                         w_ref[...].astype(jnp.bfloat16),
                         (((1,), (1,)), ((), ())),
                         preferred_element_type=jnp.float32)
    xs = jnp.sum(x * x, axis=1, keepdims=True)
    ws = ws_ref[...]
    big = jnp.int32(np.int32(2**31 - 1))
    lane_col = lax.broadcasted_iota(jnp.int32, (TOK_BLK, STRIDE), 1)
    best_v = None
    best_i = None
    for c in range(NUM_EMB // ARG_WIN):
        run_v = None
        run_i = None
        for s in range(ARG_WIN // STRIDE):
            j0 = c * ARG_WIN + s * STRIDE
            ds = (xs + ws[:, j0:j0 + STRIDE]) + m2[:, j0:j0 + STRIDE]
            if run_v is None:
                run_v = ds
                run_i = lane_col + jnp.int32(j0)
            else:
                take = ds < run_v
                run_v = jnp.where(take, ds, run_v)
                run_i = jnp.where(take, lane_col + jnp.int32(j0), run_i)
        mnc = jnp.min(run_v, axis=1, keepdims=True)
        idxc = jnp.min(jnp.where(run_v == mnc, run_i, big), axis=1, keepdims=True)
        mnc_r = mnc.astype(jnp.bfloat16).astype(jnp.float32)
        if best_v is None:
            best_v, best_i = mnc_r, idxc
        else:
            take = mnc < best_v
            best_v = jnp.where(take, mnc_r, best_v)
            best_i = jnp.where(take, idxc, best_i)
    idx_ref[0, 0, :] = best_i[:, 0]


def _k1_call(x2, w, ws2):
    n_blk = N_TOK // TOK_BLK
    out = pl.pallas_call(
        _k1_body,
        grid=(n_blk,),
        in_specs=[
            pl.BlockSpec((TOK_BLK, DIM), lambda i: (i, 0)),
            pl.BlockSpec((NUM_EMB, DIM), lambda i: (0, 0)),
            pl.BlockSpec((1, NUM_EMB), lambda i: (0, 0)),
        ],
        out_specs=pl.BlockSpec((1, 1, TOK_BLK), lambda i: (i, 0, 0)),
        out_shape=jax.ShapeDtypeStruct((n_blk, 1, TOK_BLK), jnp.int32),
    )(x2, w, ws2)
    return out.reshape(-1)


def _k2_body(idx_hbm, w_hbm, out_hbm, cnt_hbm,
             idx_v, rows_v, ones_v, zeros_v, hist_sh, sem):
    cid = lax.axis_index("c")
    sid = lax.axis_index("s")
    wid = sid * SC_NC + cid

    for i in range(CHUNK // 16):
        ones_v[pl.ds(i * 16, 16)] = jnp.ones((16,), jnp.float32)
    zchunk = NUM_EMB // SC_NS
    for i in range(zchunk // 16):
        zeros_v[pl.ds(i * 16, 16)] = jnp.zeros((16,), jnp.float32)
    pltpu.sync_copy(zeros_v, hist_sh.at[pl.ds(sid * zchunk, zchunk)])
    plsc.subcore_barrier()

    pltpu.sync_copy(idx_hbm.at[pl.ds(CHUNKS_PER_W * wid, CHUNKS_PER_W)], idx_v)
    for j in range(CHUNKS_PER_W):
        pltpu.async_copy(w_hbm.at[idx_v.at[j]], rows_v.at[j], sem).wait()
        pltpu.sync_copy(ones_v, hist_sh.at[idx_v.at[j]], add=True)
    pltpu.sync_copy(rows_v, out_hbm.at[pl.ds(CHUNKS_PER_W * wid, CHUNKS_PER_W)])

    plsc.subcore_barrier()

    @pl.when(sid == 0)
    def _():
        pltpu.sync_copy(hist_sh, cnt_hbm.at[cid])


def _k2_call(idx2, w):
    mesh = plsc.VectorSubcoreMesh(core_axis_name="c", subcore_axis_name="s")
    fn = functools.partial(
        pl.kernel,
        mesh=mesh,
        out_type=[
            jax.ShapeDtypeStruct((N_TOK // CHUNK, CHUNK, DIM), jnp.float32),
            jax.ShapeDtypeStruct((SC_NC, NUM_EMB), jnp.float32),
        ],
        scratch_types=[
            pltpu.VMEM((CHUNKS_PER_W, CHUNK), jnp.int32),
            pltpu.VMEM((CHUNKS_PER_W, CHUNK, DIM), jnp.float32),
            pltpu.VMEM((CHUNK,), jnp.float32),
            pltpu.VMEM((NUM_EMB // SC_NS,), jnp.float32),
            pltpu.VMEM_SHARED((NUM_EMB,), jnp.float32),
            pltpu.SemaphoreType.DMA,
        ],
        compiler_params=pltpu.CompilerParams(use_tc_tiling_on_sc=False),
    )(_k2_body)
    return fn(idx2, w)


def _k3_body(x_ref, q_ref, cnt_ref, qst_ref, loss_ref, perp_ref):
    x = x_ref[...]
    q = q_ref[...]
    diff = q - x
    qst_ref[...] = x + diff
    m = jnp.sum(diff * diff) * np.float32(1.0 / (N_TOK * DIM))
    loss_ref[...] = jnp.reshape(m + np.float32(COMMIT) * m, (1, 1))
    p = (cnt_ref[0:1, :] + cnt_ref[1:2, :]) * np.float32(1.0 / N_TOK)
    ent = jnp.sum(p * jnp.log(p + np.float32(EPS)))
    perp_ref[...] = jnp.reshape(jnp.exp(-ent), (1, 1))


def _k3_call(x2, q2, cnt):
    return pl.pallas_call(
        _k3_body,
        out_shape=[
            jax.ShapeDtypeStruct((N_TOK, DIM), jnp.float32),
            jax.ShapeDtypeStruct((1, 1), jnp.float32),
            jax.ShapeDtypeStruct((1, 1), jnp.float32),
        ],
    )(x2, q2, cnt)


def kernel(X, W):
    x2 = X.reshape(-1, DIM)
    ws2 = jnp.sum(W ** 2, axis=1).reshape(1, NUM_EMB)
    enc = _k1_call(x2, W, ws2)
    idx2 = enc.reshape(N_TOK // CHUNK, CHUNK)
    q_blocks, cnt = _k2_call(idx2, W)
    q2 = q_blocks.reshape(N_TOK, DIM)
    qst, loss, perp = _k3_call(x2, q2, cnt)
    quantized_st = qst.reshape(X.shape)
    enc_idx = enc.reshape(X.shape[0], -1)
    return quantized_st, enc_idx, perp.reshape(()), loss.reshape(())


# speedup vs baseline: 1.6219x; 1.0094x over previous
import functools

import numpy as np
import jax
import jax.numpy as jnp
from jax import lax
from jax.experimental import pallas as pl
from jax.experimental.pallas import tpu as pltpu
from jax.experimental.pallas import tpu_sc as plsc

NUM_EMB = 8192
DIM = 32
N_TOK = 8192
TOK_BLK = 1024
ARG_WIN = 2048
STRIDE = 128
COMMIT = 0.1
EPS = float(np.finfo(np.float32).eps)

SC_NC = 2
SC_NS = 16
SC_NW = SC_NC * SC_NS
TOK_PER_W = N_TOK // SC_NW
CHUNK = 128
CHUNKS_PER_W = TOK_PER_W // CHUNK


def _k1_body(x_ref, w_ref, ws_ref, idx_ref):
    x = x_ref[...]
    m2 = lax.dot_general((x * np.float32(-2.0)).astype(jnp.bfloat16),
                         w_ref[...].astype(jnp.bfloat16),
                         (((1,), (1,)), ((), ())),
                         preferred_element_type=jnp.float32)
    xs = jnp.sum(x * x, axis=1, keepdims=True)
    ws = ws_ref[...]
    big = jnp.int32(np.int32(2**31 - 1))
    lane_col = lax.broadcasted_iota(jnp.int32, (TOK_BLK, STRIDE), 1)
    best_v = None
    best_i = None
    for c in range(NUM_EMB // ARG_WIN):
        run_v = None
        run_i = None
        for s in range(ARG_WIN // STRIDE):
            j0 = c * ARG_WIN + s * STRIDE
            ds = (xs + ws[:, j0:j0 + STRIDE]) + m2[:, j0:j0 + STRIDE]
            if run_v is None:
                run_v = ds
                run_i = lane_col + jnp.int32(j0)
            else:
                take = ds < run_v
                run_v = jnp.where(take, ds, run_v)
                run_i = jnp.where(take, lane_col + jnp.int32(j0), run_i)
        mnc = jnp.min(run_v, axis=1, keepdims=True)
        idxc = jnp.min(jnp.where(run_v == mnc, run_i, big), axis=1, keepdims=True)
        mnc_r = mnc.astype(jnp.bfloat16).astype(jnp.float32)
        if best_v is None:
            best_v, best_i = mnc_r, idxc
        else:
            take = mnc < best_v
            best_v = jnp.where(take, mnc_r, best_v)
            best_i = jnp.where(take, idxc, best_i)
    idx_ref[0, 0, :] = best_i[:, 0]


def _k1_call(x2, w, ws2):
    n_blk = N_TOK // TOK_BLK
    out = pl.pallas_call(
        _k1_body,
        grid=(n_blk,),
        in_specs=[
            pl.BlockSpec((TOK_BLK, DIM), lambda i: (i, 0)),
            pl.BlockSpec((NUM_EMB, DIM), lambda i: (0, 0)),
            pl.BlockSpec((1, NUM_EMB), lambda i: (0, 0)),
        ],
        out_specs=pl.BlockSpec((1, 1, TOK_BLK), lambda i: (i, 0, 0)),
        out_shape=jax.ShapeDtypeStruct((n_blk, 1, TOK_BLK), jnp.int32),
    )(x2, w, ws2)
    return out.reshape(-1)


def _k2_body(idx_hbm, w_hbm, out_hbm, cnt_hbm,
             idx_v, rows_v, ones_v, zeros_v, hist_sh, sem):
    cid = lax.axis_index("c")
    sid = lax.axis_index("s")
    wid = sid * SC_NC + cid

    for i in range(CHUNK // 16):
        ones_v[pl.ds(i * 16, 16)] = jnp.ones((16,), jnp.float32)
    zchunk = NUM_EMB // SC_NS
    for i in range(zchunk // 16):
        zeros_v[pl.ds(i * 16, 16)] = jnp.zeros((16,), jnp.float32)
    pltpu.sync_copy(zeros_v, hist_sh.at[pl.ds(sid * zchunk, zchunk)])
    plsc.subcore_barrier()

    pltpu.sync_copy(idx_hbm.at[pl.ds(CHUNKS_PER_W * wid, CHUNKS_PER_W)], idx_v)
    copies = [pltpu.async_copy(w_hbm.at[idx_v.at[j]], rows_v.at[j], sem)
              for j in range(CHUNKS_PER_W)]
    for j in range(CHUNKS_PER_W):
        pltpu.sync_copy(ones_v, hist_sh.at[idx_v.at[j]], add=True)
    for c in copies:
        c.wait()
    pltpu.sync_copy(rows_v, out_hbm.at[pl.ds(CHUNKS_PER_W * wid, CHUNKS_PER_W)])

    plsc.subcore_barrier()

    @pl.when(sid == 0)
    def _():
        pltpu.sync_copy(hist_sh, cnt_hbm.at[cid])


def _k2_call(idx2, w):
    mesh = plsc.VectorSubcoreMesh(core_axis_name="c", subcore_axis_name="s")
    fn = functools.partial(
        pl.kernel,
        mesh=mesh,
        out_type=[
            jax.ShapeDtypeStruct((N_TOK // CHUNK, CHUNK, DIM), jnp.float32),
            jax.ShapeDtypeStruct((SC_NC, NUM_EMB), jnp.float32),
        ],
        scratch_types=[
            pltpu.VMEM((CHUNKS_PER_W, CHUNK), jnp.int32),
            pltpu.VMEM((CHUNKS_PER_W, CHUNK, DIM), jnp.float32),
            pltpu.VMEM((CHUNK,), jnp.float32),
            pltpu.VMEM((NUM_EMB // SC_NS,), jnp.float32),
            pltpu.VMEM_SHARED((NUM_EMB,), jnp.float32),
            pltpu.SemaphoreType.DMA,
        ],
        compiler_params=pltpu.CompilerParams(use_tc_tiling_on_sc=False),
    )(_k2_body)
    return fn(idx2, w)


def _k3_body(x_ref, q_ref, cnt_ref, qst_ref, loss_ref, perp_ref):
    x = x_ref[...]
    q = q_ref[...]
    diff = q - x
    qst_ref[...] = x + diff
    m = jnp.sum(diff * diff) * np.float32(1.0 / (N_TOK * DIM))
    loss_ref[...] = jnp.reshape(m + np.float32(COMMIT) * m, (1, 1))
    p = (cnt_ref[0:1, :] + cnt_ref[1:2, :]) * np.float32(1.0 / N_TOK)
    ent = jnp.sum(p * jnp.log(p + np.float32(EPS)))
    perp_ref[...] = jnp.reshape(jnp.exp(-ent), (1, 1))


def _k3_call(x2, q2, cnt):
    return pl.pallas_call(
        _k3_body,
        out_shape=[
            jax.ShapeDtypeStruct((N_TOK, DIM), jnp.float32),
            jax.ShapeDtypeStruct((1, 1), jnp.float32),
            jax.ShapeDtypeStruct((1, 1), jnp.float32),
        ],
    )(x2, q2, cnt)


def kernel(X, W):
    x2 = X.reshape(-1, DIM)
    ws2 = jnp.sum(W ** 2, axis=1).reshape(1, NUM_EMB)
    enc = _k1_call(x2, W, ws2)
    idx2 = enc.reshape(N_TOK // CHUNK, CHUNK)
    q_blocks, cnt = _k2_call(idx2, W)
    q2 = q_blocks.reshape(N_TOK, DIM)
    qst, loss, perp = _k3_call(x2, q2, cnt)
    quantized_st = qst.reshape(X.shape)
    enc_idx = enc.reshape(X.shape[0], -1)
    return quantized_st, enc_idx, perp.reshape(()), loss.reshape(())
